# baseline (speedup 1.0000x reference)
import functools

import jax
import jax.numpy as jnp
from jax import lax
from jax.experimental import pallas as pl
from jax.experimental.pallas import tpu as pltpu
from jax.experimental.pallas import tpu_sc as plsc

B = 4096
T = 200
C = 64
NC = 2
NS = 16
NW = NC * NS
BBLK = B // NW
NBUF = 4
LANES = 16
JSTEP = BBLK // LANES


def _body(idx_hbm, tok_hbm, pos_hbm, out_hbm, idx_v, pos_v, *rest):
    gbuf = rest[:NBUF]
    obuf = rest[NBUF:2 * NBUF]
    gsem = rest[2 * NBUF:3 * NBUF]
    ssem = rest[3 * NBUF:4 * NBUF]
    cid = lax.axis_index("c")
    sid = lax.axis_index("s")
    w = sid * NC + cid

    pltpu.sync_copy(idx_hbm.at[w], idx_v)
    pltpu.sync_copy(pos_hbm, pos_v)

    for n in range(NBUF):
        pltpu.async_copy(tok_hbm.at[idx_v.at[n]], gbuf[n], gsem[n])

    lane = lax.iota(jnp.int32, LANES)

    def item(t, carry):
        for n in range(NBUF):
            tt = NBUF * t + n

            @pl.when(tt >= NBUF)
            def _():
                pltpu.make_async_copy(
                    obuf[n], out_hbm.at[0, pl.ds(0, C), pl.ds(0, BBLK)],
                    ssem[n]).wait()

            pltpu.make_async_copy(
                tok_hbm.at[idx_v.at[tt]], gbuf[n], gsem[n]).wait()

            def col(c, c2, _n=n, _tt=tt):
                cc = jnp.full((LANES,), c, jnp.int32)
                psplat = plsc.load_gather(
                    pos_v, [jnp.full((LANES,), _tt, jnp.int32), cc])
                for j in range(JSTEP):
                    rows = jnp.full((LANES,), j * LANES, jnp.int32) + lane
                    v = plsc.load_gather(gbuf[_n], [rows, cc])
                    obuf[_n][c, pl.ds(j * LANES, LANES)] = v + psplat
                return c2

            lax.fori_loop(0, C, col, 0)

            pltpu.async_copy(
                obuf[n],
                out_hbm.at[tt, pl.ds(0, C), pl.ds(w * BBLK, BBLK)],
                ssem[n])

            @pl.when(tt < T - NBUF)
            def _():
                pltpu.async_copy(
                    tok_hbm.at[idx_v.at[tt + NBUF]], gbuf[n], gsem[n])
        return carry

    lax.fori_loop(0, T // NBUF, item, 0)

    for n in range(NBUF):
        pltpu.make_async_copy(
            obuf[n], out_hbm.at[0, pl.ds(0, C), pl.ds(0, BBLK)],
            ssem[n]).wait()


def _run(idx3, tok, pos):
    mesh = plsc.VectorSubcoreMesh(core_axis_name="c", subcore_axis_name="s")
    k = functools.partial(
        pl.kernel,
        mesh=mesh,
        out_type=jax.ShapeDtypeStruct((T, C, B), jnp.float32),
        scratch_types=(
            [pltpu.VMEM((T, BBLK), jnp.int32),
             pltpu.VMEM((T, C), jnp.float32)]
            + [pltpu.VMEM((BBLK, C), jnp.float32) for _ in range(NBUF)]
            + [pltpu.VMEM((C, BBLK), jnp.float32) for _ in range(NBUF)]
            + [pltpu.SemaphoreType.DMA for _ in range(2 * NBUF)]
        ),
        compiler_params=pltpu.CompilerParams(
            use_tc_tiling_on_sc=False, needs_layout_passes=False),
    )(_body)
    return k(idx3, tok, pos)


def kernel(idx, token_embedding_table, position_embedding_table):
    idx3 = jnp.transpose(
        idx.astype(jnp.int32).reshape(NW, BBLK, T), (0, 2, 1))
    res = _run(idx3, token_embedding_table, position_embedding_table)
    return jnp.transpose(res, (2, 0, 1))


# speedup vs baseline: 1.0860x; 1.0860x over previous
import functools

import jax
import jax.numpy as jnp
from jax import lax
from jax.experimental import pallas as pl
from jax.experimental.pallas import tpu as pltpu
from jax.experimental.pallas import tpu_sc as plsc

B = 4096
T = 200
C = 64
NC = 2
NS = 16
NW = NC * NS
BBLK = B // NW
NBUF = 4
LANES = 16
JSTEP = BBLK // LANES
CA = C // 8


def _body(idx_hbm, tok_hbm, pos_hbm, out_hbm, idx_v, pos_v, *rest):
    gbuf = rest[:NBUF]
    obuf = rest[NBUF:2 * NBUF]
    gsem = rest[2 * NBUF:3 * NBUF]
    ssem = rest[3 * NBUF:4 * NBUF]
    cid = lax.axis_index("c")
    sid = lax.axis_index("s")
    w = sid * NC + cid

    pltpu.sync_copy(idx_hbm.at[w], idx_v)
    pltpu.sync_copy(pos_hbm, pos_v)

    for n in range(NBUF):
        pltpu.async_copy(tok_hbm.at[idx_v.at[n]], gbuf[n], gsem[n])

    lane = lax.iota(jnp.int32, LANES)

    def item(t, carry):
        for n in range(NBUF):
            tt = NBUF * t + n

            @pl.when(tt >= NBUF)
            def _():
                for a in range(CA):
                    pltpu.make_async_copy(
                        obuf[n].at[a], out_hbm.at[0, a, pl.ds(0, 8)],
                        ssem[n]).wait()

            pltpu.make_async_copy(
                tok_hbm.at[idx_v.at[tt]], gbuf[n], gsem[n]).wait()

            def suba(a, c2, _n=n, _tt=tt):
                for ci in range(8):
                    c = 8 * a + ci
                    cc = jnp.full((LANES,), c, jnp.int32)
                    psplat = plsc.load_gather(
                        pos_v, [jnp.full((LANES,), _tt, jnp.int32), cc])
                    for j in range(JSTEP):
                        rows = jnp.full((LANES,), j * LANES, jnp.int32) + lane
                        v = plsc.load_gather(gbuf[_n], [rows, cc])
                        obuf[_n][a, ci, pl.ds(j * LANES, LANES)] = v + psplat
                return c2

            lax.fori_loop(0, CA, suba, 0)

            for a in range(CA):
                pltpu.async_copy(
                    obuf[n].at[a],
                    out_hbm.at[tt, a, pl.ds(w * 8, 8)],
                    ssem[n])

            @pl.when(tt < T - NBUF)
            def _():
                pltpu.async_copy(
                    tok_hbm.at[idx_v.at[tt + NBUF]], gbuf[n], gsem[n])
        return carry

    lax.fori_loop(0, T // NBUF, item, 0)

    for n in range(NBUF):
        for a in range(CA):
            pltpu.make_async_copy(
                obuf[n].at[a], out_hbm.at[0, a, pl.ds(0, 8)],
                ssem[n]).wait()


def _run(idx3, tok, pos):
    mesh = plsc.VectorSubcoreMesh(core_axis_name="c", subcore_axis_name="s")
    k = functools.partial(
        pl.kernel,
        mesh=mesh,
        out_type=jax.ShapeDtypeStruct((T, CA, NW * 8, BBLK), jnp.float32),
        scratch_types=(
            [pltpu.VMEM((T, BBLK), jnp.int32),
             pltpu.VMEM((T, C), jnp.float32)]
            + [pltpu.VMEM((BBLK, C), jnp.float32) for _ in range(NBUF)]
            + [pltpu.VMEM((CA, 8, BBLK), jnp.float32) for _ in range(NBUF)]
            + [pltpu.SemaphoreType.DMA for _ in range(2 * NBUF)]
        ),
        compiler_params=pltpu.CompilerParams(
            use_tc_tiling_on_sc=False, needs_layout_passes=False),
    )(_body)
    return k(idx3, tok, pos)


def kernel(idx, token_embedding_table, position_embedding_table):
    idx3 = jnp.transpose(
        idx.astype(jnp.int32).reshape(NW, BBLK, T), (0, 2, 1))
    res = _run(idx3, token_embedding_table, position_embedding_table)
    res5 = res.reshape(T, CA, NW, 8, BBLK)
    return jnp.transpose(res5, (2, 4, 0, 1, 3)).reshape(B, T, C)


# speedup vs baseline: 1.9429x; 1.7891x over previous
import functools

import jax
import jax.numpy as jnp
from jax import lax
from jax.experimental import pallas as pl
from jax.experimental.pallas import tpu as pltpu
from jax.experimental.pallas import tpu_sc as plsc

B = 4096
T = 200
C = 64
NC = 2
NS = 16
NW = NC * NS
BBLK = B // NW
NBUF = 4
LANES = 16
JSTEP = BBLK // LANES
CA = C // 8


def _body(idx_hbm, tok_hbm, pos_hbm, out_hbm, idx_v, pos_v, *rest):
    gbuf = rest[:NBUF]
    obuf = rest[NBUF:2 * NBUF]
    gsem = rest[2 * NBUF:3 * NBUF]
    ssem = rest[3 * NBUF:4 * NBUF]
    cid = lax.axis_index("c")
    sid = lax.axis_index("s")
    w = sid * NC + cid

    pltpu.sync_copy(idx_hbm.at[w], idx_v)
    pltpu.sync_copy(pos_hbm, pos_v)

    for n in range(NBUF):
        pltpu.async_copy(tok_hbm.at[idx_v.at[n]], gbuf[n], gsem[n])

    lane = lax.iota(jnp.int32, LANES)

    def item(t, carry):
        for n in range(NBUF):
            tt = NBUF * t + n

            @pl.when(tt >= NBUF)
            def _():
                for a in range(CA):
                    pltpu.make_async_copy(
                        obuf[n].at[a], out_hbm.at[0, a, pl.ds(0, 8)],
                        ssem[n]).wait()

            pltpu.make_async_copy(
                tok_hbm.at[idx_v.at[tt]], gbuf[n], gsem[n]).wait()

            def rblock(rb, c2, _n=n, _tt=tt):
                r0 = rb * LANES
                for cb in range(C // LANES):
                    cols = lane + cb * LANES
                    a_idx = lax.shift_right_logical(cols, 3)
                    ci_idx = lax.bitwise_and(cols, 7)
                    pvec = pos_v[_tt, pl.ds(cb * LANES, LANES)]
                    for k in range(LANES):
                        rows = r0 + lax.bitwise_and(lane + k, 15)
                        v = plsc.load_gather(gbuf[_n], [rows, cols])
                        plsc.store_scatter(
                            obuf[_n], [a_idx, ci_idx, rows], v + pvec)
                return c2

            lax.fori_loop(0, JSTEP, rblock, 0)

            for a in range(CA):
                pltpu.async_copy(
                    obuf[n].at[a],
                    out_hbm.at[tt, a, pl.ds(w * 8, 8)],
                    ssem[n])

            @pl.when(tt < T - NBUF)
            def _():
                pltpu.async_copy(
                    tok_hbm.at[idx_v.at[tt + NBUF]], gbuf[n], gsem[n])
        return carry

    lax.fori_loop(0, T // NBUF, item, 0)

    for n in range(NBUF):
        for a in range(CA):
            pltpu.make_async_copy(
                obuf[n].at[a], out_hbm.at[0, a, pl.ds(0, 8)],
                ssem[n]).wait()


def _run(idx3, tok, pos):
    mesh = plsc.VectorSubcoreMesh(core_axis_name="c", subcore_axis_name="s")
    k = functools.partial(
        pl.kernel,
        mesh=mesh,
        out_type=jax.ShapeDtypeStruct((T, CA, NW * 8, BBLK), jnp.float32),
        scratch_types=(
            [pltpu.VMEM((T, BBLK), jnp.int32),
             pltpu.VMEM((T, C), jnp.float32)]
            + [pltpu.VMEM((BBLK, C), jnp.float32) for _ in range(NBUF)]
            + [pltpu.VMEM((CA, 8, BBLK), jnp.float32) for _ in range(NBUF)]
            + [pltpu.SemaphoreType.DMA for _ in range(2 * NBUF)]
        ),
        compiler_params=pltpu.CompilerParams(
            use_tc_tiling_on_sc=False, needs_layout_passes=False),
    )(_body)
    return k(idx3, tok, pos)


def kernel(idx, token_embedding_table, position_embedding_table):
    idx3 = jnp.transpose(
        idx.astype(jnp.int32).reshape(NW, BBLK, T), (0, 2, 1))
    res = _run(idx3, token_embedding_table, position_embedding_table)
    res5 = res.reshape(T, CA, NW, 8, BBLK)
    return jnp.transpose(res5, (2, 4, 0, 1, 3)).reshape(B, T, C)


# speedup vs baseline: 1.9473x; 1.0023x over previous
import functools

import jax
import jax.numpy as jnp
from jax import lax
from jax.experimental import pallas as pl
from jax.experimental.pallas import tpu as pltpu
from jax.experimental.pallas import tpu_sc as plsc

B = 4096
T = 200
C = 64
VOCAB = 1000000
NC = 2
NS = 16
NW = NC * NS
BBLK = B // NW
NBUF = 4
LANES = 16
JSTEP = BBLK // LANES
CA = C // 8


def _body(idx_hbm, tok_hbm, pos_hbm, out_hbm, idx_v, pos_v, *rest):
    gbuf = rest[:NBUF]
    obuf = rest[NBUF:2 * NBUF]
    gsem = rest[2 * NBUF:3 * NBUF]
    ssem = rest[3 * NBUF:4 * NBUF]
    cid = lax.axis_index("c")
    sid = lax.axis_index("s")
    w = sid * NC + cid

    pltpu.sync_copy(idx_hbm.at[w], idx_v)
    pltpu.sync_copy(pos_hbm, pos_v)

    for n in range(NBUF):
        pltpu.async_copy(tok_hbm.at[idx_v.at[n]], gbuf[n], gsem[n])

    lane = lax.iota(jnp.int32, LANES)

    def item(t, carry):
        for n in range(NBUF):
            tt = NBUF * t + n

            @pl.when(tt >= NBUF)
            def _():
                for a in range(CA):
                    pltpu.make_async_copy(
                        obuf[n].at[a], out_hbm.at[0, a, pl.ds(0, 8)],
                        ssem[n]).wait()

            pltpu.make_async_copy(
                tok_hbm.at[idx_v.at[tt]], gbuf[n], gsem[n]).wait()

            def rblock(rb, c2, _n=n, _tt=tt):
                r0 = rb * LANES
                for cb in range(C // LANES):
                    cols = lane + cb * LANES
                    a_idx = lax.shift_right_logical(cols, 3)
                    ci_idx = lax.bitwise_and(cols, 7)
                    pvec = pos_v[_tt, pl.ds(cb * LANES, LANES)]
                    for k in range(LANES):
                        rows = r0 + lax.bitwise_and(lane + k, 15)
                        v = plsc.load_gather(gbuf[_n], [rows, cols])
                        plsc.store_scatter(
                            obuf[_n], [a_idx, ci_idx, rows], v + pvec)
                return c2

            lax.fori_loop(0, JSTEP, rblock, 0)

            for a in range(CA):
                pltpu.async_copy(
                    obuf[n].at[a],
                    out_hbm.at[tt, a, pl.ds(w * 8, 8)],
                    ssem[n])

            @pl.when(tt < T - NBUF)
            def _():
                pltpu.async_copy(
                    tok_hbm.at[idx_v.at[tt + NBUF]], gbuf[n], gsem[n])
        return carry

    lax.fori_loop(0, T // NBUF, item, 0)

    for n in range(NBUF):
        for a in range(CA):
            pltpu.make_async_copy(
                obuf[n].at[a], out_hbm.at[0, a, pl.ds(0, 8)],
                ssem[n]).wait()


def _run(idx3, tok, pos):
    mesh = plsc.VectorSubcoreMesh(core_axis_name="c", subcore_axis_name="s")
    k = functools.partial(
        pl.kernel,
        mesh=mesh,
        out_type=jax.ShapeDtypeStruct((T, CA, NW * 8, BBLK), jnp.float32),
        scratch_types=(
            [pltpu.VMEM((T, BBLK), jnp.int32),
             pltpu.VMEM((T, C), jnp.float32)]
            + [pltpu.VMEM((BBLK, C), jnp.float32) for _ in range(NBUF)]
            + [pltpu.VMEM((CA, 8, BBLK), jnp.float32) for _ in range(NBUF)]
            + [pltpu.SemaphoreType.DMA for _ in range(2 * NBUF)]
        ),
        compiler_params=pltpu.CompilerParams(
            use_tc_tiling_on_sc=False, needs_layout_passes=False),
    )(_body)
    return k(idx3, tok, pos)


def kernel(idx, token_embedding_table, position_embedding_table):
    idx3 = jnp.transpose(
        idx.astype(jnp.int32).reshape(NW, BBLK, T), (0, 2, 1))
    t2 = lax.optimization_barrier(
        jnp.reshape(token_embedding_table, (VOCAB // 2, 2 * C)))
    tab = jnp.reshape(t2, (VOCAB, C))
    res = _run(idx3, tab, position_embedding_table)
    res5 = res.reshape(T, CA, NW, 8, BBLK)
    return jnp.transpose(res5, (2, 4, 0, 1, 3)).reshape(B, T, C)
